# Initial kernel scaffold; baseline (speedup 1.0000x reference)
import jax
import jax.numpy as jnp
from jax.experimental import pallas as pl


def kernel(adj_indices, adj_values, num_nodes, h, pos_u, pos_v, neg_u, neg_v, W1, b1, W2, b2, Wd1, bd1, Wd2, bd2):
    raise NotImplementedError("write your pallas kernel here")


# speedup vs baseline: 4.1487x; 4.1487x over previous
import jax
import jax.numpy as jnp
from jax import lax
from jax.experimental import pallas as pl
from jax.experimental.pallas import tpu as pltpu
from jax.experimental.pallas import tpu_sc as plsc

NC = 2
NS = 16
NW = NC * NS
L = 16
K = 128


def _make_spmm(n_pad, d_feat, e_pad, with_deg):
    chunks_per_tile = e_pad // (NW * K)
    nj = d_feat // L
    seg = n_pad // NS
    mesh = plsc.VectorSubcoreMesh(core_axis_name="c", subcore_axis_name="s")

    out_type = [jax.ShapeDtypeStruct((NC, n_pad, d_feat), jnp.float32)]
    scratch = [
        pltpu.VMEM((K,), jnp.int32),
        pltpu.VMEM((K,), jnp.int32),
        pltpu.VMEM((K,), jnp.float32),
        pltpu.VMEM((K, d_feat), jnp.float32),
        pltpu.VMEM_SHARED((n_pad, d_feat), jnp.float32),
        pltpu.SemaphoreType.DMA,
    ]
    if with_deg:
        out_type.append(jax.ShapeDtypeStruct((NC, n_pad), jnp.float32))
        scratch += [
            pltpu.VMEM((n_pad,), jnp.float32),
            pltpu.VMEM((seg,), jnp.float32),
            pltpu.VMEM((seg,), jnp.float32),
            pltpu.VMEM_SHARED((NS * n_pad,), jnp.float32),
        ]

    def body(rows_h, cols_h, vals_h, x_h, znd_h, out_h, *rest):
        if with_deg:
            deg_h, idxc, idxr, valb, gath, acc, sem, dtile, dtmp, dloc, dsh = rest
        else:
            (idxc, idxr, valb, gath, acc, sem) = rest
        c = lax.axis_index("c")
        s = lax.axis_index("s")
        zv = jnp.zeros((L,), jnp.float32)
        lane0 = lax.iota(jnp.int32, L) == 0

        @pl.when(s == 0)
        def _init():
            pltpu.sync_copy(znd_h, acc)

        if with_deg:
            def zinit(i, carry):
                dtile[pl.ds(i * L, L)] = zv
                return carry
            lax.fori_loop(0, n_pad // L, zinit, 0)

        plsc.subcore_barrier()

        tile_base = (c * NS + s) * (chunks_per_tile * K)

        def chunk(g, carry):
            base = tile_base + g * K
            pltpu.sync_copy(cols_h.at[pl.ds(base, K)], idxc)
            pltpu.sync_copy(rows_h.at[pl.ds(base, K)], idxr)
            pltpu.sync_copy(vals_h.at[pl.ds(base, K)], valb)
            pltpu.async_copy(x_h.at[idxc], gath, sem).wait()

            def edge(e, carry2):
                fe = jnp.full((L,), e, jnp.int32)
                vv = plsc.load_gather(valb, [fe])
                for j in range(nj):
                    gath[e, pl.ds(j * L, L)] = gath[e, pl.ds(j * L, L)] * vv
                if with_deg:
                    rr = plsc.load_gather(idxr, [fe])
                    plsc.addupdate_scatter(dtile, [rr], vv, mask=lane0)
                return carry2

            lax.fori_loop(0, K, edge, 0)
            pltpu.sync_copy(gath, acc.at[idxr], add=True)
            return carry

        lax.fori_loop(0, chunks_per_tile, chunk, 0)
        if with_deg:
            pltpu.sync_copy(dtile, dsh.at[pl.ds(s * n_pad, n_pad)])
        plsc.subcore_barrier()

        rpt = n_pad // NS
        r0 = s * rpt
        pltpu.sync_copy(acc.at[pl.ds(r0, rpt)],
                        out_h.at[c, pl.ds(r0, rpt)])

        if with_deg:
            d0 = s * seg
            def z2(i, carry):
                dloc[pl.ds(i * L, L)] = zv
                return carry
            lax.fori_loop(0, seg // L, z2, 0)

            def comb(tt, carry):
                pltpu.sync_copy(dsh.at[pl.ds(tt * n_pad + d0, seg)], dtmp)
                for i in range(seg // L):
                    dloc[pl.ds(i * L, L)] = (dloc[pl.ds(i * L, L)]
                                             + dtmp[pl.ds(i * L, L)])
                return carry
            lax.fori_loop(0, NS, comb, 0)
            pltpu.sync_copy(dloc, deg_h.at[c, pl.ds(d0, seg)])

    return pl.kernel(body, out_type=out_type, mesh=mesh,
                     scratch_types=scratch,
                     compiler_params=pltpu.CompilerParams(
                         needs_layout_passes=False))


def _make_decode(n, d_feat, t_pairs):
    pairs_per_tile = t_pairs // NW
    nj = d_feat // L
    mesh = plsc.VectorSubcoreMesh(core_axis_name="c", subcore_axis_name="s")

    out_type = jax.ShapeDtypeStruct((t_pairs,), jnp.float32)
    scratch = [
        pltpu.VMEM((K,), jnp.int32),
        pltpu.VMEM((K,), jnp.int32),
        pltpu.VMEM((K, d_feat), jnp.float32),
        pltpu.VMEM((K, d_feat), jnp.float32),
        pltpu.VMEM((L, L), jnp.float32),
        pltpu.VMEM((d_feat,), jnp.float32),
        pltpu.VMEM((L,), jnp.float32),
        pltpu.VMEM((K,), jnp.float32),
        pltpu.SemaphoreType.DMA,
        pltpu.SemaphoreType.DMA,
    ]
    chunks_per_tile = pairs_per_tile // K

    def body(a_h, b_h, u_h, v_h, w2_h, bd2_h, out_h,
             idxu, idxv, ga, gb, tbuf, wbuf, bdbuf, sbuf, sem1, sem2):
        c = lax.axis_index("c")
        s = lax.axis_index("s")
        pltpu.sync_copy(w2_h, wbuf)
        pltpu.sync_copy(bd2_h, bdbuf)
        tile_base = (c * NS + s) * pairs_per_tile
        iota16 = lax.iota(jnp.int32, L)

        def chunk(g, carry):
            base = tile_base + g * K
            pltpu.sync_copy(u_h.at[pl.ds(base, K)], idxu)
            pltpu.sync_copy(v_h.at[pl.ds(base, K)], idxv)
            pltpu.async_copy(a_h.at[idxu], ga, sem1).wait()
            pltpu.async_copy(b_h.at[idxv], gb, sem2).wait()

            def group(q, carry2):
                for k in range(L):
                    p = q * L + k
                    accv = jnp.zeros((L,), jnp.float32)
                    for j in range(nj):
                        t = jnp.maximum(
                            ga[p, pl.ds(j * L, L)] + gb[p, pl.ds(j * L, L)],
                            0.0)
                        accv = accv + t * wbuf[pl.ds(j * L, L)]
                    tbuf[k, :] = accv
                sv = bdbuf[:]
                for l in range(L):
                    sv = sv + plsc.load_gather(
                        tbuf, [iota16, jnp.full((L,), l, jnp.int32)])
                sbuf[pl.ds(q * L, L)] = sv
                return carry2

            lax.fori_loop(0, K // L, group, 0)
            pltpu.sync_copy(sbuf, out_h.at[pl.ds(base, K)])
            return carry

        lax.fori_loop(0, chunks_per_tile, chunk, 0)

    return pl.kernel(body, out_type=out_type, mesh=mesh,
                     scratch_types=scratch,
                     compiler_params=pltpu.CompilerParams(
                         needs_layout_passes=False))


def _stage_a_body(acc_ref, dacc_ref, w_ref, b_ref, zd_ref, o_ref):
    x = acc_ref[0] + acc_ref[1]
    deg = dacc_ref[0] + dacc_ref[1] + zd_ref[0, 0]
    di = 1.0 / jnp.maximum(deg, 1.0)
    y = lax.dot_general(x * di[:, None], w_ref[...],
                        (((1,), (1,)), ((), ())),
                        preferred_element_type=jnp.float32)
    o_ref[...] = jnp.maximum(y + b_ref[...], 0.0)


def _stage_b_body(acc_ref, dacc_ref, w2_ref, b2_ref, wdu_ref, wdv_ref,
                  bd1_ref, zd_ref, h2_ref, a_ref, bmat_ref):
    x = acc_ref[0] + acc_ref[1]
    deg = dacc_ref[0] + dacc_ref[1] + zd_ref[0, 0]
    di = 1.0 / jnp.maximum(deg, 1.0)
    h2 = lax.dot_general(x * di[:, None], w2_ref[...],
                         (((1,), (1,)), ((), ())),
                         preferred_element_type=jnp.float32) + b2_ref[...]
    h2_ref[...] = h2
    a_ref[...] = lax.dot_general(h2, wdu_ref[...], (((1,), (1,)), ((), ())),
                                 preferred_element_type=jnp.float32) + bd1_ref[...]
    bmat_ref[...] = lax.dot_general(h2, wdv_ref[...], (((1,), (1,)), ((), ())),
                                    preferred_element_type=jnp.float32)


def _dense_stage_a(acc, dacc, w1, b1, zd, bn=1024):
    n = acc.shape[1]
    d = acc.shape[2]
    grid = (n // bn,)
    return pl.pallas_call(
        _stage_a_body,
        grid=grid,
        in_specs=[
            pl.BlockSpec((NC, bn, d), lambda i: (0, i, 0)),
            pl.BlockSpec((NC, bn), lambda i: (0, i)),
            pl.BlockSpec((d, d), lambda i: (0, 0)),
            pl.BlockSpec((1, d), lambda i: (0, 0)),
            pl.BlockSpec((1, 1), lambda i: (0, 0)),
        ],
        out_specs=pl.BlockSpec((bn, d), lambda i: (i, 0)),
        out_shape=jax.ShapeDtypeStruct((n, d), jnp.float32),
    )(acc, dacc, w1, b1, zd)


def _dense_stage_b(acc, dacc, w2, b2, wdu, wdv, bd1, zd, bn=1024):
    n = acc.shape[1]
    d = acc.shape[2]
    grid = (n // bn,)
    mat = jax.ShapeDtypeStruct((n, d), jnp.float32)
    return pl.pallas_call(
        _stage_b_body,
        grid=grid,
        in_specs=[
            pl.BlockSpec((NC, bn, d), lambda i: (0, i, 0)),
            pl.BlockSpec((NC, bn), lambda i: (0, i)),
            pl.BlockSpec((d, d), lambda i: (0, 0)),
            pl.BlockSpec((1, d), lambda i: (0, 0)),
            pl.BlockSpec((d, d), lambda i: (0, 0)),
            pl.BlockSpec((d, d), lambda i: (0, 0)),
            pl.BlockSpec((1, d), lambda i: (0, 0)),
            pl.BlockSpec((1, 1), lambda i: (0, 0)),
        ],
        out_specs=[
            pl.BlockSpec((bn, d), lambda i: (i, 0)),
            pl.BlockSpec((bn, d), lambda i: (i, 0)),
            pl.BlockSpec((bn, d), lambda i: (i, 0)),
        ],
        out_shape=[mat, mat, mat],
    )(acc, dacc, w2, b2, wdu, wdv, bd1, zd)


def kernel(adj_indices, adj_values, num_nodes, h, pos_u, pos_v, neg_u, neg_v,
           W1, b1, W2, b2, Wd1, bd1, Wd2, bd2):
    n, d_feat = h.shape
    e = adj_values.shape[0]
    p = pos_u.shape[0]

    e_pad = ((e + NW * K - 1) // (NW * K)) * (NW * K)
    pad = e_pad - e
    rows = adj_indices[:, 0].astype(jnp.int32)
    cols = adj_indices[:, 1].astype(jnp.int32)
    vals = adj_values
    if pad:
        fill = (jnp.arange(pad, dtype=jnp.int32) * 97) % n
        rows = jnp.concatenate([rows, fill])
        cols = jnp.concatenate([cols, fill])
        vals = jnp.concatenate([vals, jnp.zeros((pad,), jnp.float32)])

    n_pad = ((n + NS * L - 1) // (NS * L)) * (NS * L)
    if n_pad % 1024:
        n_pad = ((n_pad + 1023) // 1024) * 1024
    znd = jnp.zeros((n_pad, d_feat), jnp.float32)
    zd = jnp.reshape(
        (jnp.asarray(num_nodes) - n).astype(jnp.float32), (1, 1))

    spmm_deg = _make_spmm(n_pad, d_feat, e_pad, with_deg=True)
    spmm = _make_spmm(n_pad, d_feat, e_pad, with_deg=False)

    acc1, dacc = spmm_deg(rows, cols, vals, h, znd)
    h1 = _dense_stage_a(acc1, dacc, W1, b1.reshape(1, -1), zd)
    (acc2,) = spmm(rows, cols, vals, h1, znd)
    h2, a_mat, b_mat = _dense_stage_b(
        acc2, dacc, W2, b2.reshape(1, -1), Wd1[:, :d_feat], Wd1[:, d_feat:],
        bd1.reshape(1, -1), zd)

    u_all = jnp.concatenate([pos_u, neg_u]).astype(jnp.int32)
    v_all = jnp.concatenate([pos_v, neg_v]).astype(jnp.int32)
    w2vec = Wd2[0]
    bd2b = jnp.full((L,), bd2[0], jnp.float32)

    decode = _make_decode(n_pad, d_feat, 2 * p)
    scores = decode(a_mat, b_mat, u_all, v_all, w2vec, bd2b)
    return (scores[:p], scores[p:], h2[:n])


# speedup vs baseline: 6.2927x; 1.5168x over previous
import jax
import jax.numpy as jnp
from jax import lax
from jax.experimental import pallas as pl
from jax.experimental.pallas import tpu as pltpu
from jax.experimental.pallas import tpu_sc as plsc

NC = 2
NS = 16
NW = NC * NS
L = 16
K = 128


def _make_spmm(n_pad, d_feat, e_pad, with_deg):
    chunks_per_tile = e_pad // (NW * K)
    nj = d_feat // L
    mesh = plsc.VectorSubcoreMesh(core_axis_name="c", subcore_axis_name="s")

    out_type = [jax.ShapeDtypeStruct((NC, n_pad, d_feat), jnp.float32)]
    scratch = [
        pltpu.VMEM((2, K), jnp.int32),
        pltpu.VMEM((2, K), jnp.int32),
        pltpu.VMEM((2, K), jnp.float32),
        pltpu.VMEM((2, K, d_feat), jnp.float32),
        pltpu.VMEM_SHARED((n_pad, d_feat), jnp.float32),
        pltpu.SemaphoreType.DMA,
        pltpu.SemaphoreType.DMA,
        pltpu.SemaphoreType.DMA,
        pltpu.SemaphoreType.DMA,
        pltpu.SemaphoreType.DMA,
    ]
    if with_deg:
        out_type.append(jax.ShapeDtypeStruct((NC, NS, n_pad), jnp.float32))
        scratch += [
            pltpu.VMEM((n_pad,), jnp.float32),
        ]

    def body(rows_h, cols_h, vals_h, x_h, znd_h, out_h, *rest):
        if with_deg:
            (deg_h, idxc, idxr, valb, gath, acc, isem,
             gsem0, gsem1, ssem0, ssem1, dtile) = rest
        else:
            (idxc, idxr, valb, gath, acc, isem,
             gsem0, gsem1, ssem0, ssem1) = rest
        gsem = (gsem0, gsem1)
        ssem = (ssem0, ssem1)
        c = lax.axis_index("c")
        s = lax.axis_index("s")
        zv = jnp.zeros((L,), jnp.float32)
        lane0 = lax.iota(jnp.int32, L) == 0
        G = chunks_per_tile
        tile_base = (c * NS + s) * (G * K)

        def load_idx(g, slot):
            base = tile_base + g * K
            d1 = pltpu.async_copy(cols_h.at[pl.ds(base, K)], idxc.at[slot], isem)
            d2 = pltpu.async_copy(rows_h.at[pl.ds(base, K)], idxr.at[slot], isem)
            d3 = pltpu.async_copy(vals_h.at[pl.ds(base, K)], valb.at[slot], isem)
            d1.wait()
            d2.wait()
            d3.wait()

        load_idx(0, 0)
        pltpu.async_copy(x_h.at[idxc.at[0]], gath.at[0], gsem[0])

        @pl.when(s == 0)
        def _init():
            pltpu.sync_copy(znd_h, acc)

        if with_deg:
            def zinit(i, carry):
                dtile[pl.ds(i * L, L)] = zv
                return carry
            lax.fori_loop(0, n_pad // L, zinit, 0)

        plsc.subcore_barrier()

        def pair(gg, carry):
            for b in range(2):
                g = gg * 2 + b
                nb = 1 - b
                pltpu.make_async_copy(
                    x_h.at[idxc.at[b]], gath.at[b], gsem[b]).wait()

                @pl.when(g + 1 < G)
                def _prep():
                    @pl.when(g >= 1)
                    def _drain():
                        pltpu.make_async_copy(
                            gath.at[nb], acc.at[idxr.at[nb]], ssem[nb]).wait()
                    load_idx(g + 1, nb)
                    pltpu.async_copy(
                        x_h.at[idxc.at[nb]], gath.at[nb], gsem[nb])

                def edge(e, carry2):
                    fe = jnp.full((L,), e, jnp.int32)
                    vv = plsc.load_gather(valb.at[b], [fe])
                    for j in range(nj):
                        gath[b, e, pl.ds(j * L, L)] = (
                            gath[b, e, pl.ds(j * L, L)] * vv)
                    if with_deg:
                        rr = plsc.load_gather(idxr.at[b], [fe])
                        plsc.addupdate_scatter(dtile, [rr], vv, mask=lane0)
                    return carry2

                lax.fori_loop(0, K, edge, 0)
                pltpu.async_copy(
                    gath.at[b], acc.at[idxr.at[b]], ssem[b], add=True)
            return carry

        lax.fori_loop(0, G // 2, pair, 0)
        pltpu.make_async_copy(gath.at[0], acc.at[idxr.at[0]], ssem[0]).wait()
        pltpu.make_async_copy(gath.at[1], acc.at[idxr.at[1]], ssem[1]).wait()
        if with_deg:
            pltpu.sync_copy(dtile, deg_h.at[c, s])
        plsc.subcore_barrier()

        rpt = n_pad // NS
        r0 = s * rpt
        pltpu.sync_copy(acc.at[pl.ds(r0, rpt)],
                        out_h.at[c, pl.ds(r0, rpt)])


    return pl.kernel(body, out_type=out_type, mesh=mesh,
                     scratch_types=scratch,
                     compiler_params=pltpu.CompilerParams(
                         needs_layout_passes=False))


def _make_decode(n, d_feat, t_pairs):
    pairs_per_tile = t_pairs // NW
    nj = d_feat // L
    mesh = plsc.VectorSubcoreMesh(core_axis_name="c", subcore_axis_name="s")

    out_type = jax.ShapeDtypeStruct((t_pairs,), jnp.float32)
    scratch = [
        pltpu.VMEM((K,), jnp.int32),
        pltpu.VMEM((K,), jnp.int32),
        pltpu.VMEM((K, d_feat), jnp.float32),
        pltpu.VMEM((K, d_feat), jnp.float32),
        pltpu.VMEM((L, L), jnp.float32),
        pltpu.VMEM((d_feat,), jnp.float32),
        pltpu.VMEM((L,), jnp.float32),
        pltpu.VMEM((K,), jnp.float32),
        pltpu.SemaphoreType.DMA,
        pltpu.SemaphoreType.DMA,
    ]
    chunks_per_tile = pairs_per_tile // K

    def body(a_h, b_h, u_h, v_h, w2_h, bd2_h, out_h,
             idxu, idxv, ga, gb, tbuf, wbuf, bdbuf, sbuf, sem1, sem2):
        c = lax.axis_index("c")
        s = lax.axis_index("s")
        pltpu.sync_copy(w2_h, wbuf)
        pltpu.sync_copy(bd2_h, bdbuf)
        tile_base = (c * NS + s) * pairs_per_tile
        iota16 = lax.iota(jnp.int32, L)

        def chunk(g, carry):
            base = tile_base + g * K
            pltpu.sync_copy(u_h.at[pl.ds(base, K)], idxu)
            pltpu.sync_copy(v_h.at[pl.ds(base, K)], idxv)
            pltpu.async_copy(a_h.at[idxu], ga, sem1).wait()
            pltpu.async_copy(b_h.at[idxv], gb, sem2).wait()

            def group(q, carry2):
                for k in range(L):
                    p = q * L + k
                    accv = jnp.zeros((L,), jnp.float32)
                    for j in range(nj):
                        t = jnp.maximum(
                            ga[p, pl.ds(j * L, L)] + gb[p, pl.ds(j * L, L)],
                            0.0)
                        accv = accv + t * wbuf[pl.ds(j * L, L)]
                    tbuf[k, :] = accv
                sv = bdbuf[:]
                for l in range(L):
                    sv = sv + plsc.load_gather(
                        tbuf, [iota16, jnp.full((L,), l, jnp.int32)])
                sbuf[pl.ds(q * L, L)] = sv
                return carry2

            lax.fori_loop(0, K // L, group, 0)
            pltpu.sync_copy(sbuf, out_h.at[pl.ds(base, K)])
            return carry

        lax.fori_loop(0, chunks_per_tile, chunk, 0)

    return pl.kernel(body, out_type=out_type, mesh=mesh,
                     scratch_types=scratch,
                     compiler_params=pltpu.CompilerParams(
                         needs_layout_passes=False))


def _stage_a_body(acc_ref, dacc_ref, w_ref, b_ref, zd_ref, o_ref):
    x = acc_ref[0] + acc_ref[1]
    deg = jnp.sum(dacc_ref[...], axis=(0, 1)) + zd_ref[0, 0]
    di = 1.0 / jnp.maximum(deg, 1.0)
    y = lax.dot_general(x * di[:, None], w_ref[...],
                        (((1,), (1,)), ((), ())),
                        preferred_element_type=jnp.float32)
    o_ref[...] = jnp.maximum(y + b_ref[...], 0.0)


def _stage_b_body(acc_ref, dacc_ref, w2_ref, b2_ref, wdu_ref, wdv_ref,
                  bd1_ref, zd_ref, h2_ref, a_ref, bmat_ref):
    x = acc_ref[0] + acc_ref[1]
    deg = jnp.sum(dacc_ref[...], axis=(0, 1)) + zd_ref[0, 0]
    di = 1.0 / jnp.maximum(deg, 1.0)
    h2 = lax.dot_general(x * di[:, None], w2_ref[...],
                         (((1,), (1,)), ((), ())),
                         preferred_element_type=jnp.float32) + b2_ref[...]
    h2_ref[...] = h2
    a_ref[...] = lax.dot_general(h2, wdu_ref[...], (((1,), (1,)), ((), ())),
                                 preferred_element_type=jnp.float32) + bd1_ref[...]
    bmat_ref[...] = lax.dot_general(h2, wdv_ref[...], (((1,), (1,)), ((), ())),
                                    preferred_element_type=jnp.float32)


def _dense_stage_a(acc, dacc, w1, b1, zd, bn=1024):
    n = acc.shape[1]
    d = acc.shape[2]
    grid = (n // bn,)
    return pl.pallas_call(
        _stage_a_body,
        grid=grid,
        in_specs=[
            pl.BlockSpec((NC, bn, d), lambda i: (0, i, 0)),
            pl.BlockSpec((NC, NS, bn), lambda i: (0, 0, i)),
            pl.BlockSpec((d, d), lambda i: (0, 0)),
            pl.BlockSpec((1, d), lambda i: (0, 0)),
            pl.BlockSpec((1, 1), lambda i: (0, 0)),
        ],
        out_specs=pl.BlockSpec((bn, d), lambda i: (i, 0)),
        out_shape=jax.ShapeDtypeStruct((n, d), jnp.float32),
    )(acc, dacc, w1, b1, zd)


def _dense_stage_b(acc, dacc, w2, b2, wdu, wdv, bd1, zd, bn=1024):
    n = acc.shape[1]
    d = acc.shape[2]
    grid = (n // bn,)
    mat = jax.ShapeDtypeStruct((n, d), jnp.float32)
    return pl.pallas_call(
        _stage_b_body,
        grid=grid,
        in_specs=[
            pl.BlockSpec((NC, bn, d), lambda i: (0, i, 0)),
            pl.BlockSpec((NC, NS, bn), lambda i: (0, 0, i)),
            pl.BlockSpec((d, d), lambda i: (0, 0)),
            pl.BlockSpec((1, d), lambda i: (0, 0)),
            pl.BlockSpec((d, d), lambda i: (0, 0)),
            pl.BlockSpec((d, d), lambda i: (0, 0)),
            pl.BlockSpec((1, d), lambda i: (0, 0)),
            pl.BlockSpec((1, 1), lambda i: (0, 0)),
        ],
        out_specs=[
            pl.BlockSpec((bn, d), lambda i: (i, 0)),
            pl.BlockSpec((bn, d), lambda i: (i, 0)),
            pl.BlockSpec((bn, d), lambda i: (i, 0)),
        ],
        out_shape=[mat, mat, mat],
    )(acc, dacc, w2, b2, wdu, wdv, bd1, zd)


def kernel(adj_indices, adj_values, num_nodes, h, pos_u, pos_v, neg_u, neg_v,
           W1, b1, W2, b2, Wd1, bd1, Wd2, bd2):
    n, d_feat = h.shape
    e = adj_values.shape[0]
    p = pos_u.shape[0]

    e_pad = ((e + 2 * NW * K - 1) // (2 * NW * K)) * (2 * NW * K)
    pad = e_pad - e
    rows = adj_indices[:, 0].astype(jnp.int32)
    cols = adj_indices[:, 1].astype(jnp.int32)
    vals = adj_values
    if pad:
        fill = (jnp.arange(pad, dtype=jnp.int32) * 97) % n
        rows = jnp.concatenate([rows, fill])
        cols = jnp.concatenate([cols, fill])
        vals = jnp.concatenate([vals, jnp.zeros((pad,), jnp.float32)])

    n_pad = ((n + NS * L - 1) // (NS * L)) * (NS * L)
    if n_pad % 1024:
        n_pad = ((n_pad + 1023) // 1024) * 1024
    znd = jnp.zeros((n_pad, d_feat), jnp.float32)
    zd = jnp.reshape(
        (jnp.asarray(num_nodes) - n).astype(jnp.float32), (1, 1))

    spmm_deg = _make_spmm(n_pad, d_feat, e_pad, with_deg=True)
    spmm = _make_spmm(n_pad, d_feat, e_pad, with_deg=False)

    acc1, dacc = spmm_deg(rows, cols, vals, h, znd)
    h1 = _dense_stage_a(acc1, dacc, W1, b1.reshape(1, -1), zd)
    (acc2,) = spmm(rows, cols, vals, h1, znd)
    h2, a_mat, b_mat = _dense_stage_b(
        acc2, dacc, W2, b2.reshape(1, -1), Wd1[:, :d_feat], Wd1[:, d_feat:],
        bd1.reshape(1, -1), zd)

    u_all = jnp.concatenate([pos_u, neg_u]).astype(jnp.int32)
    v_all = jnp.concatenate([pos_v, neg_v]).astype(jnp.int32)
    w2vec = Wd2[0]
    bd2b = jnp.full((L,), bd2[0], jnp.float32)

    decode = _make_decode(n_pad, d_feat, 2 * p)
    scores = decode(a_mat, b_mat, u_all, v_all, w2vec, bd2b)
    return (scores[:p], scores[p:], h2[:n])
